# Initial kernel scaffold; baseline (speedup 1.0000x reference)
import jax
import jax.numpy as jnp
from jax.experimental import pallas as pl


def kernel(x, edge_index, W1, b1, W2, b2):
    raise NotImplementedError("write your pallas kernel here")


# speedup vs baseline: 1.1379x; 1.1379x over previous
import functools

import jax
import jax.numpy as jnp
from jax import lax
from jax.experimental import pallas as pl
from jax.experimental.pallas import tpu as pltpu
from jax.experimental.pallas import tpu_sc as plsc

NC = 2
NS = 16
NW = NC * NS
L = 16
CH = 128
NSEG = 16


def _mesh():
    return plsc.VectorSubcoreMesh(core_axis_name="c", subcore_axis_name="s",
                                  num_cores=NC, num_subcores=NS)


def _slab_rows(N):
    R = -(-N // NW)
    R = -(-R // L) * L
    return R


def _make_prep(N, E):
    R = _slab_rows(N)
    SEG = E // NSEG
    M = (SEG + CH - 1) // CH + 1
    CAP = M * CH

    @functools.partial(
        pl.kernel,
        out_type=(
            jax.ShapeDtypeStruct((N,), jnp.float32),
            jax.ShapeDtypeStruct((NW * NSEG * CAP,), jnp.int32),
            jax.ShapeDtypeStruct((NW * NSEG * CAP,), jnp.int32),
            jax.ShapeDtypeStruct((NW * NSEG * L,), jnp.int32),
        ),
        mesh=_mesh(),
        compiler_params=pltpu.CompilerParams(needs_layout_passes=False),
        scratch_types=[
            pltpu.VMEM((SEG,), jnp.int32),
            pltpu.VMEM((SEG,), jnp.int32),
            pltpu.VMEM((CAP,), jnp.int32),
            pltpu.VMEM((CAP,), jnp.int32),
            pltpu.VMEM((R + L,), jnp.float32),
            pltpu.VMEM((L,), jnp.int32),
        ],
    )
    def prep(src_hbm, dst_hbm, deg_hbm, srcc_hbm, dstc_hbm, cnt_hbm,
             sbuf, dbuf, scb, dcb, part, cntb):
        c = lax.axis_index("c")
        s = lax.axis_index("s")
        wid = c * NS + s
        base = wid * R

        zf = jnp.zeros((L,), jnp.float32)

        def zero_part(i, _):
            part[pl.ds(i * L, L)] = zf
            return 0
        lax.fori_loop(0, (R + L) // L, zero_part, 0)

        zi = jnp.zeros((L,), jnp.int32)
        dummy = jnp.full((L,), R, jnp.int32)
        ones = jnp.ones((L,), jnp.float32)

        def seg_body(k, _):
            pltpu.sync_copy(src_hbm.at[pl.ds(k * SEG, SEG)], sbuf)
            pltpu.sync_copy(dst_hbm.at[pl.ds(k * SEG, SEG)], dbuf)

            def prefill(i, _):
                scb[pl.ds(i * L, L)] = zi
                dcb[pl.ds(i * L, L)] = dummy
                return 0
            lax.fori_loop(0, CAP // L, prefill, 0)

            def scan_body(j, cnt):
                sv = sbuf[pl.ds(j * L, L)]
                dv = dbuf[pl.ds(j * L, L)]
                loc = dv - base
                mask = (loc >= 0) & (loc < R)
                locc = jnp.where(mask, loc, R)
                plsc.addupdate_scatter(part, [locc], ones, mask=mask)
                plsc.store_compressed(scb.at[pl.ds(cnt, L)], sv, mask=mask)
                plsc.store_compressed(dcb.at[pl.ds(cnt, L)], locc, mask=mask)
                return cnt + jnp.sum(mask.astype(jnp.int32))
            cnt = lax.fori_loop(0, SEG // L, scan_body, jnp.int32(0))

            cntb[...] = jnp.full((L,), cnt, jnp.int32)
            pltpu.sync_copy(cntb, cnt_hbm.at[pl.ds((wid * NSEG + k) * L, L)])
            pltpu.sync_copy(scb, srcc_hbm.at[pl.ds((wid * NSEG + k) * CAP, CAP)])
            pltpu.sync_copy(dcb, dstc_hbm.at[pl.ds((wid * NSEG + k) * CAP, CAP)])
            return 0
        lax.fori_loop(0, NSEG, seg_body, 0)

        def deg_body(i, _):
            part[pl.ds(i * L, L)] = part[pl.ds(i * L, L)] + 1.0
            return 0
        lax.fori_loop(0, R // L, deg_body, 0)

        rows = N - (NW - 1) * R

        @pl.when(wid < NW - 1)
        def _():
            pltpu.sync_copy(part.at[pl.ds(0, R)], deg_hbm.at[pl.ds(base, R)])

        @pl.when(wid == NW - 1)
        def _():
            pltpu.sync_copy(part.at[pl.ds(0, rows)], deg_hbm.at[pl.ds(base, rows)])

    return prep


def _make_edge(N, D, E):
    R = _slab_rows(N)
    SEG = E // NSEG
    M = (SEG + CH - 1) // CH + 1
    CAP = M * CH
    SLAB = R + 8

    @functools.partial(
        pl.kernel,
        out_type=jax.ShapeDtypeStruct((N, D), jnp.float32),
        mesh=_mesh(),
        compiler_params=pltpu.CompilerParams(needs_layout_passes=False),
        scratch_types=[
            pltpu.VMEM((CH,), jnp.int32),
            pltpu.VMEM((CH,), jnp.int32),
            pltpu.VMEM((NW * NSEG * L,), jnp.int32),
            pltpu.VMEM((CH, D), jnp.float32),
            pltpu.VMEM((SLAB, D), jnp.float32),
            pltpu.SemaphoreType.DMA,
        ],
    )
    def edge(hs_hbm, srcc_hbm, dstc_hbm, cnt_hbm, out_hbm,
             idxs, idxd, cb, gbuf, slab, sem):
        c = lax.axis_index("c")
        s = lax.axis_index("s")
        wid = c * NS + s
        base = wid * R
        rows = N - (NW - 1) * R

        pltpu.sync_copy(cnt_hbm, cb)

        @pl.when(wid < NW - 1)
        def _():
            pltpu.sync_copy(hs_hbm.at[pl.ds(base, R)], slab.at[pl.ds(0, R)])

        @pl.when(wid == NW - 1)
        def _():
            pltpu.sync_copy(hs_hbm.at[pl.ds(base, rows)], slab.at[pl.ds(0, rows)])

        lanes = jnp.arange(L, dtype=jnp.int32)

        def seg_body(k, _):
            cnt = cb[pl.ds((wid * NSEG + k) * L, L)][0]
            nch = (cnt + CH - 1) // CH

            def chunk(t, _):
                ebase = (wid * NSEG + k) * CAP + t * CH
                pltpu.sync_copy(srcc_hbm.at[pl.ds(ebase, CH)], idxs)
                pltpu.sync_copy(dstc_hbm.at[pl.ds(ebase, CH)], idxd)
                pltpu.async_copy(hs_hbm.at[idxs], gbuf, sem).wait()

                def group(g, _):
                    dv = idxd[pl.ds(g * L, L)]
                    rv = g * L + lanes
                    cv = jnp.zeros((L,), jnp.int32)
                    onev = jnp.ones((L,), jnp.int32)
                    for col in range(D):
                        v = plsc.load_gather(gbuf, [rv, cv])
                        plsc.addupdate_scatter(slab, [dv, cv], v)
                        cv = cv + onev
                    return 0
                lax.fori_loop(0, CH // L, group, 0)
                return 0
            lax.fori_loop(0, nch, chunk, 0)
            return 0
        lax.fori_loop(0, NSEG, seg_body, 0)

        @pl.when(wid < NW - 1)
        def _():
            pltpu.sync_copy(slab.at[pl.ds(0, R)], out_hbm.at[pl.ds(base, R)])

        @pl.when(wid == NW - 1)
        def _():
            pltpu.sync_copy(slab.at[pl.ds(0, rows)], out_hbm.at[pl.ds(base, rows)])

    return edge


def _layer1_tc(x, W, b, deg, bm):
    N, D = x.shape

    def body(x_ref, w_ref, b_ref, deg_ref, hs_ref, dis_ref):
        dis = lax.rsqrt(deg_ref[...])
        h = jnp.dot(x_ref[...], w_ref[...],
                    preferred_element_type=jnp.float32) + b_ref[...]
        hs_ref[...] = h * dis
        dis_ref[...] = dis

    return pl.pallas_call(
        body,
        grid=(N // bm,),
        in_specs=[
            pl.BlockSpec((bm, D), lambda i: (i, 0)),
            pl.BlockSpec((D, D), lambda i: (0, 0)),
            pl.BlockSpec((1, D), lambda i: (0, 0)),
            pl.BlockSpec((bm, 1), lambda i: (i, 0)),
        ],
        out_specs=[
            pl.BlockSpec((bm, D), lambda i: (i, 0)),
            pl.BlockSpec((bm, 1), lambda i: (i, 0)),
        ],
        out_shape=[
            jax.ShapeDtypeStruct((N, D), jnp.float32),
            jax.ShapeDtypeStruct((N, 1), jnp.float32),
        ],
    )(x, W, b, deg)


def _layer2_tc(acc1, dis, W, b, bm):
    N, D = acc1.shape

    def body(a_ref, dis_ref, w_ref, b_ref, hs_ref):
        dis = dis_ref[...]
        h_in = jnp.maximum(a_ref[...] * dis, 0.0)
        h = jnp.dot(h_in, w_ref[...],
                    preferred_element_type=jnp.float32) + b_ref[...]
        hs_ref[...] = h * dis

    return pl.pallas_call(
        body,
        grid=(N // bm,),
        in_specs=[
            pl.BlockSpec((bm, D), lambda i: (i, 0)),
            pl.BlockSpec((bm, 1), lambda i: (i, 0)),
            pl.BlockSpec((D, D), lambda i: (0, 0)),
            pl.BlockSpec((1, D), lambda i: (0, 0)),
        ],
        out_specs=pl.BlockSpec((bm, D), lambda i: (i, 0)),
        out_shape=jax.ShapeDtypeStruct((N, D), jnp.float32),
    )(acc1, dis, W, b)


def _scale_tc(acc2, dis, bm):
    N, D = acc2.shape

    def body(a_ref, dis_ref, o_ref):
        o_ref[...] = a_ref[...] * dis_ref[...]

    return pl.pallas_call(
        body,
        grid=(N // bm,),
        in_specs=[
            pl.BlockSpec((bm, D), lambda i: (i, 0)),
            pl.BlockSpec((bm, 1), lambda i: (i, 0)),
        ],
        out_specs=pl.BlockSpec((bm, D), lambda i: (i, 0)),
        out_shape=jax.ShapeDtypeStruct((N, D), jnp.float32),
    )(acc2, dis)


def kernel(x, edge_index, W1, b1, W2, b2):
    N, D = x.shape
    E = edge_index.shape[1]
    assert E % (NSEG * L) == 0 and D % L == 0
    bm = 1000 if N % 1000 == 0 else 8

    src = edge_index[0].astype(jnp.int32)
    dst = edge_index[1].astype(jnp.int32)

    deg, srcc, dstc, counts = _make_prep(N, E)(src, dst)
    deg = deg.reshape(N, 1)

    b1r = b1.reshape(1, D)
    b2r = b2.reshape(1, D)

    edge_fn = _make_edge(N, D, E)

    hs1, dis = _layer1_tc(x, W1, b1r, deg, bm)
    acc1 = edge_fn(hs1, srcc, dstc, counts)
    hs2 = _layer2_tc(acc1, dis, W2, b2r, bm)
    acc2 = edge_fn(hs2, srcc, dstc, counts)
    return _scale_tc(acc2, dis, bm)


# speedup vs baseline: 1.9436x; 1.7081x over previous
import functools

import jax
import jax.numpy as jnp
from jax import lax
from jax.experimental import pallas as pl
from jax.experimental.pallas import tpu as pltpu
from jax.experimental.pallas import tpu_sc as plsc

NC = 2
NS = 16
NW = NC * NS
L = 16
CH = 128
NSEG = 16


def _mesh():
    return plsc.VectorSubcoreMesh(core_axis_name="c", subcore_axis_name="s",
                                  num_cores=NC, num_subcores=NS)


def _slab_rows(N):
    R = -(-N // NW)
    R = -(-R // L) * L
    return R


def _make_prep(N, E):
    R = _slab_rows(N)
    SEG = E // NSEG
    M = (SEG + CH - 1) // CH + 1
    CAP = M * CH

    @functools.partial(
        pl.kernel,
        out_type=(
            jax.ShapeDtypeStruct((N,), jnp.float32),
            jax.ShapeDtypeStruct((NW * NSEG * CAP,), jnp.int32),
            jax.ShapeDtypeStruct((NW * NSEG * CAP,), jnp.int32),
            jax.ShapeDtypeStruct((NW * NSEG * L,), jnp.int32),
        ),
        mesh=_mesh(),
        compiler_params=pltpu.CompilerParams(needs_layout_passes=False),
        scratch_types=[
            pltpu.VMEM((SEG,), jnp.int32),
            pltpu.VMEM((SEG,), jnp.int32),
            pltpu.VMEM((CAP,), jnp.int32),
            pltpu.VMEM((CAP,), jnp.int32),
            pltpu.VMEM((R + L,), jnp.float32),
            pltpu.VMEM((L,), jnp.int32),
        ],
    )
    def prep(src_hbm, dst_hbm, deg_hbm, srcc_hbm, dstc_hbm, cnt_hbm,
             sbuf, dbuf, scb, dcb, part, cntb):
        c = lax.axis_index("c")
        s = lax.axis_index("s")
        wid = c * NS + s
        base = wid * R

        zf = jnp.zeros((L,), jnp.float32)

        def zero_part(i, _):
            part[pl.ds(i * L, L)] = zf
            return 0
        lax.fori_loop(0, (R + L) // L, zero_part, 0)

        zi = jnp.zeros((L,), jnp.int32)
        dummy = jnp.full((L,), R, jnp.int32)
        ones = jnp.ones((L,), jnp.float32)

        def seg_body(k, _):
            pltpu.sync_copy(src_hbm.at[pl.ds(k * SEG, SEG)], sbuf)
            pltpu.sync_copy(dst_hbm.at[pl.ds(k * SEG, SEG)], dbuf)

            def prefill(i, _):
                scb[pl.ds(i * L, L)] = zi
                dcb[pl.ds(i * L, L)] = dummy
                return 0
            lax.fori_loop(0, CAP // L, prefill, 0)

            def scan_body(j, cnt):
                sv = sbuf[pl.ds(j * L, L)]
                dv = dbuf[pl.ds(j * L, L)]
                loc = dv - base
                mask = (loc >= 0) & (loc < R)
                locc = jnp.where(mask, loc, R)
                plsc.addupdate_scatter(part, [locc], ones, mask=mask)
                plsc.store_compressed(scb.at[pl.ds(cnt, L)], sv, mask=mask)
                plsc.store_compressed(dcb.at[pl.ds(cnt, L)], locc, mask=mask)
                return cnt + jnp.sum(mask.astype(jnp.int32))
            cnt = lax.fori_loop(0, SEG // L, scan_body, jnp.int32(0))

            cntb[...] = jnp.full((L,), cnt, jnp.int32)
            pltpu.sync_copy(cntb, cnt_hbm.at[pl.ds((wid * NSEG + k) * L, L)])
            pltpu.sync_copy(scb, srcc_hbm.at[pl.ds((wid * NSEG + k) * CAP, CAP)])
            pltpu.sync_copy(dcb, dstc_hbm.at[pl.ds((wid * NSEG + k) * CAP, CAP)])
            return 0
        lax.fori_loop(0, NSEG, seg_body, 0)

        def deg_body(i, _):
            part[pl.ds(i * L, L)] = part[pl.ds(i * L, L)] + 1.0
            return 0
        lax.fori_loop(0, R // L, deg_body, 0)

        rows = N - (NW - 1) * R

        @pl.when(wid < NW - 1)
        def _():
            pltpu.sync_copy(part.at[pl.ds(0, R)], deg_hbm.at[pl.ds(base, R)])

        @pl.when(wid == NW - 1)
        def _():
            pltpu.sync_copy(part.at[pl.ds(0, rows)], deg_hbm.at[pl.ds(base, rows)])

    return prep


def _make_edge(N, D, E):
    R = _slab_rows(N)
    SEG = E // NSEG
    M = (SEG + CH - 1) // CH + 1
    CAP = M * CH
    SLAB = R + 8

    @functools.partial(
        pl.kernel,
        out_type=jax.ShapeDtypeStruct((N * D,), jnp.float32),
        mesh=_mesh(),
        compiler_params=pltpu.CompilerParams(needs_layout_passes=False),
        scratch_types=[
            pltpu.VMEM((CH,), jnp.int32),
            pltpu.VMEM((CH,), jnp.int32),
            pltpu.VMEM((NW * NSEG * L,), jnp.int32),
            pltpu.VMEM((CH, D), jnp.float32),
            pltpu.VMEM((SLAB * D,), jnp.float32),
            pltpu.SemaphoreType.DMA,
        ],
    )
    def edge(hs_hbm, hsf_hbm, srcc_hbm, dstc_hbm, cnt_hbm, out_hbm,
             idxs, idxd, cb, gbuf, slab, sem):
        c = lax.axis_index("c")
        s = lax.axis_index("s")
        wid = c * NS + s
        base = wid * R
        rows = N - (NW - 1) * R

        pltpu.sync_copy(cnt_hbm, cb)

        @pl.when(wid < NW - 1)
        def _():
            pltpu.sync_copy(hsf_hbm.at[pl.ds(base * D, R * D)],
                            slab.at[pl.ds(0, R * D)])

        @pl.when(wid == NW - 1)
        def _():
            pltpu.sync_copy(hsf_hbm.at[pl.ds(base * D, rows * D)],
                            slab.at[pl.ds(0, rows * D)])

        def seg_body(k, _):
            cnt = cb[pl.ds((wid * NSEG + k) * L, L)][0]
            nch = (cnt + CH - 1) // CH

            def chunk(t, _):
                ebase = (wid * NSEG + k) * CAP + t * CH
                pltpu.sync_copy(srcc_hbm.at[pl.ds(ebase, CH)], idxs)
                pltpu.sync_copy(dstc_hbm.at[pl.ds(ebase, CH)], idxd)
                pltpu.async_copy(hs_hbm.at[idxs], gbuf, sem).wait()

                def group(g, _):
                    dvec = idxd[pl.ds(g * L, L)] * D
                    for l in range(L):
                        rbase = dvec[l]
                        grow = g * L + l
                        for j in range(D // L):
                            plsc.addupdate(slab.at[pl.ds(rbase + j * L, L)],
                                           gbuf[grow, pl.ds(j * L, L)])
                    return 0
                lax.fori_loop(0, CH // L, group, 0)
                return 0
            lax.fori_loop(0, nch, chunk, 0)
            return 0
        lax.fori_loop(0, NSEG, seg_body, 0)

        @pl.when(wid < NW - 1)
        def _():
            pltpu.sync_copy(slab.at[pl.ds(0, R * D)],
                            out_hbm.at[pl.ds(base * D, R * D)])

        @pl.when(wid == NW - 1)
        def _():
            pltpu.sync_copy(slab.at[pl.ds(0, rows * D)],
                            out_hbm.at[pl.ds(base * D, rows * D)])

    return edge


def _layer1_tc(x, W, b, deg, bm):
    N, D = x.shape

    def body(x_ref, w_ref, b_ref, deg_ref, hs_ref, dis_ref):
        dis = lax.rsqrt(deg_ref[...])
        h = jnp.dot(x_ref[...], w_ref[...],
                    preferred_element_type=jnp.float32) + b_ref[...]
        hs_ref[...] = h * dis
        dis_ref[...] = dis

    return pl.pallas_call(
        body,
        grid=(N // bm,),
        in_specs=[
            pl.BlockSpec((bm, D), lambda i: (i, 0)),
            pl.BlockSpec((D, D), lambda i: (0, 0)),
            pl.BlockSpec((1, D), lambda i: (0, 0)),
            pl.BlockSpec((bm, 1), lambda i: (i, 0)),
        ],
        out_specs=[
            pl.BlockSpec((bm, D), lambda i: (i, 0)),
            pl.BlockSpec((bm, 1), lambda i: (i, 0)),
        ],
        out_shape=[
            jax.ShapeDtypeStruct((N, D), jnp.float32),
            jax.ShapeDtypeStruct((N, 1), jnp.float32),
        ],
    )(x, W, b, deg)


def _layer2_tc(acc1, dis, W, b, bm):
    N, D = acc1.shape

    def body(a_ref, dis_ref, w_ref, b_ref, hs_ref):
        dis = dis_ref[...]
        h_in = jnp.maximum(a_ref[...] * dis, 0.0)
        h = jnp.dot(h_in, w_ref[...],
                    preferred_element_type=jnp.float32) + b_ref[...]
        hs_ref[...] = h * dis

    return pl.pallas_call(
        body,
        grid=(N // bm,),
        in_specs=[
            pl.BlockSpec((bm, D), lambda i: (i, 0)),
            pl.BlockSpec((bm, 1), lambda i: (i, 0)),
            pl.BlockSpec((D, D), lambda i: (0, 0)),
            pl.BlockSpec((1, D), lambda i: (0, 0)),
        ],
        out_specs=pl.BlockSpec((bm, D), lambda i: (i, 0)),
        out_shape=jax.ShapeDtypeStruct((N, D), jnp.float32),
    )(acc1, dis, W, b)


def _scale_tc(acc2, dis, bm):
    N, D = acc2.shape

    def body(a_ref, dis_ref, o_ref):
        o_ref[...] = a_ref[...] * dis_ref[...]

    return pl.pallas_call(
        body,
        grid=(N // bm,),
        in_specs=[
            pl.BlockSpec((bm, D), lambda i: (i, 0)),
            pl.BlockSpec((bm, 1), lambda i: (i, 0)),
        ],
        out_specs=pl.BlockSpec((bm, D), lambda i: (i, 0)),
        out_shape=jax.ShapeDtypeStruct((N, D), jnp.float32),
    )(acc2, dis)


def kernel(x, edge_index, W1, b1, W2, b2):
    N, D = x.shape
    E = edge_index.shape[1]
    assert E % (NSEG * L) == 0 and D % L == 0
    bm = 1000 if N % 1000 == 0 else 8

    src = edge_index[0].astype(jnp.int32)
    dst = edge_index[1].astype(jnp.int32)

    deg, srcc, dstc, counts = _make_prep(N, E)(src, dst)
    deg = deg.reshape(N, 1)

    b1r = b1.reshape(1, D)
    b2r = b2.reshape(1, D)

    edge_fn = _make_edge(N, D, E)

    hs1, dis = _layer1_tc(x, W1, b1r, deg, bm)
    acc1 = edge_fn(hs1, hs1.reshape(-1), srcc, dstc, counts).reshape(N, D)
    hs2 = _layer2_tc(acc1, dis, W2, b2r, bm)
    acc2 = edge_fn(hs2, hs2.reshape(-1), srcc, dstc, counts).reshape(N, D)
    return _scale_tc(acc2, dis, bm)


# speedup vs baseline: 1.9470x; 1.0017x over previous
import functools

import jax
import jax.numpy as jnp
from jax import lax
from jax.experimental import pallas as pl
from jax.experimental.pallas import tpu as pltpu
from jax.experimental.pallas import tpu_sc as plsc

NC = 2
NS = 16
NW = NC * NS
L = 16
CH = 128
NSEG = 16


def _mesh():
    return plsc.VectorSubcoreMesh(core_axis_name="c", subcore_axis_name="s",
                                  num_cores=NC, num_subcores=NS)


def _slab_rows(N):
    R = -(-N // NW)
    R = -(-R // L) * L
    return R


def _make_prep(N, E):
    R = _slab_rows(N)
    SEG = E // NSEG
    M = (SEG + CH - 1) // CH + 1
    CAP = M * CH

    @functools.partial(
        pl.kernel,
        out_type=(
            jax.ShapeDtypeStruct((N,), jnp.float32),
            jax.ShapeDtypeStruct((NW * NSEG * CAP,), jnp.int32),
            jax.ShapeDtypeStruct((NW * NSEG * CAP,), jnp.int32),
            jax.ShapeDtypeStruct((NW * NSEG * L,), jnp.int32),
        ),
        mesh=_mesh(),
        compiler_params=pltpu.CompilerParams(needs_layout_passes=False),
        scratch_types=[
            pltpu.VMEM((SEG,), jnp.int32),
            pltpu.VMEM((SEG,), jnp.int32),
            pltpu.VMEM((CAP,), jnp.int32),
            pltpu.VMEM((CAP,), jnp.int32),
            pltpu.VMEM((R + L,), jnp.float32),
            pltpu.VMEM((L,), jnp.int32),
        ],
    )
    def prep(src_hbm, dst_hbm, deg_hbm, srcc_hbm, dstc_hbm, cnt_hbm,
             sbuf, dbuf, scb, dcb, part, cntb):
        c = lax.axis_index("c")
        s = lax.axis_index("s")
        wid = c * NS + s
        base = wid * R

        zf = jnp.zeros((L,), jnp.float32)

        def zero_part(i, _):
            part[pl.ds(i * L, L)] = zf
            return 0
        lax.fori_loop(0, (R + L) // L, zero_part, 0)

        zi = jnp.zeros((L,), jnp.int32)
        dummy = jnp.full((L,), R, jnp.int32)
        ones = jnp.ones((L,), jnp.float32)

        def seg_body(k, _):
            pltpu.sync_copy(src_hbm.at[pl.ds(k * SEG, SEG)], sbuf)
            pltpu.sync_copy(dst_hbm.at[pl.ds(k * SEG, SEG)], dbuf)

            def prefill(i, _):
                scb[pl.ds(i * L, L)] = zi
                dcb[pl.ds(i * L, L)] = dummy
                return 0
            lax.fori_loop(0, CAP // L, prefill, 0)

            def scan_body(j, cnt):
                sv = sbuf[pl.ds(j * L, L)]
                dv = dbuf[pl.ds(j * L, L)]
                loc = dv - base
                mask = (loc >= 0) & (loc < R)
                locc = jnp.where(mask, loc, R)
                plsc.addupdate_scatter(part, [locc], ones, mask=mask)
                plsc.store_compressed(scb.at[pl.ds(cnt, L)], sv, mask=mask)
                plsc.store_compressed(dcb.at[pl.ds(cnt, L)], locc, mask=mask)
                return cnt + jnp.sum(mask.astype(jnp.int32))
            cnt = lax.fori_loop(0, SEG // L, scan_body, jnp.int32(0))

            cntb[...] = jnp.full((L,), cnt, jnp.int32)
            pltpu.sync_copy(cntb, cnt_hbm.at[pl.ds((wid * NSEG + k) * L, L)])
            pltpu.sync_copy(scb, srcc_hbm.at[pl.ds((wid * NSEG + k) * CAP, CAP)])
            pltpu.sync_copy(dcb, dstc_hbm.at[pl.ds((wid * NSEG + k) * CAP, CAP)])
            return 0
        lax.fori_loop(0, NSEG, seg_body, 0)

        def deg_body(i, _):
            part[pl.ds(i * L, L)] = part[pl.ds(i * L, L)] + 1.0
            return 0
        lax.fori_loop(0, R // L, deg_body, 0)

        rows = N - (NW - 1) * R

        @pl.when(wid < NW - 1)
        def _():
            pltpu.sync_copy(part.at[pl.ds(0, R)], deg_hbm.at[pl.ds(base, R)])

        @pl.when(wid == NW - 1)
        def _():
            pltpu.sync_copy(part.at[pl.ds(0, rows)], deg_hbm.at[pl.ds(base, rows)])

    return prep


def _make_edge(N, D, E):
    R = _slab_rows(N)
    SEG = E // NSEG
    M = (SEG + CH - 1) // CH + 1
    CAP = M * CH
    SLAB = R + 8

    @functools.partial(
        pl.kernel,
        out_type=jax.ShapeDtypeStruct((N * D,), jnp.float32),
        mesh=_mesh(),
        compiler_params=pltpu.CompilerParams(needs_layout_passes=False),
        scratch_types=[
            pltpu.VMEM((CH,), jnp.int32),
            pltpu.VMEM((CH,), jnp.int32),
            pltpu.VMEM((NW * NSEG * L,), jnp.int32),
            pltpu.VMEM((CH, D), jnp.float32),
            pltpu.VMEM((SLAB * D,), jnp.float32),
            pltpu.SemaphoreType.DMA,
            pltpu.SemaphoreType.DMA,
        ],
    )
    def edge(hs_hbm, hsf_hbm, srcc_hbm, dstc_hbm, cnt_hbm, out_hbm,
             idxs, idxd, cb, gbuf, slab, sem, sem2):
        c = lax.axis_index("c")
        s = lax.axis_index("s")
        wid = c * NS + s
        base = wid * R
        rows = N - (NW - 1) * R

        pltpu.sync_copy(cnt_hbm, cb)

        @pl.when(wid < NW - 1)
        def _():
            pltpu.sync_copy(hsf_hbm.at[pl.ds(base * D, R * D)],
                            slab.at[pl.ds(0, R * D)])

        @pl.when(wid == NW - 1)
        def _():
            pltpu.sync_copy(hsf_hbm.at[pl.ds(base * D, rows * D)],
                            slab.at[pl.ds(0, rows * D)])

        def seg_body(k, _):
            cnt = cb[pl.ds((wid * NSEG + k) * L, L)][0]
            nch = (cnt + CH - 1) // CH

            def chunk(t, _):
                ebase = (wid * NSEG + k) * CAP + t * CH
                d1 = pltpu.async_copy(srcc_hbm.at[pl.ds(ebase, CH)], idxs, sem2)
                d2 = pltpu.async_copy(dstc_hbm.at[pl.ds(ebase, CH)], idxd, sem2)
                d1.wait()
                d2.wait()
                pltpu.async_copy(hs_hbm.at[idxs], gbuf, sem).wait()

                def group(g, _):
                    dvec = idxd[pl.ds(g * L, L)] * D
                    for l in range(L):
                        rbase = dvec[l]
                        grow = g * L + l
                        for j in range(D // L):
                            plsc.addupdate(slab.at[pl.ds(rbase + j * L, L)],
                                           gbuf[grow, pl.ds(j * L, L)])
                    return 0
                lax.fori_loop(0, CH // L, group, 0)
                return 0
            lax.fori_loop(0, nch, chunk, 0)
            return 0
        lax.fori_loop(0, NSEG, seg_body, 0)

        @pl.when(wid < NW - 1)
        def _():
            pltpu.sync_copy(slab.at[pl.ds(0, R * D)],
                            out_hbm.at[pl.ds(base * D, R * D)])

        @pl.when(wid == NW - 1)
        def _():
            pltpu.sync_copy(slab.at[pl.ds(0, rows * D)],
                            out_hbm.at[pl.ds(base * D, rows * D)])

    return edge


def _layer1_tc(x, W, b, deg, bm):
    N, D = x.shape

    def body(x_ref, w_ref, b_ref, deg_ref, hs_ref, dis_ref):
        dis = lax.rsqrt(deg_ref[...])
        h = jnp.dot(x_ref[...], w_ref[...],
                    preferred_element_type=jnp.float32) + b_ref[...]
        hs_ref[...] = h * dis
        dis_ref[...] = dis

    return pl.pallas_call(
        body,
        grid=(N // bm,),
        in_specs=[
            pl.BlockSpec((bm, D), lambda i: (i, 0)),
            pl.BlockSpec((D, D), lambda i: (0, 0)),
            pl.BlockSpec((1, D), lambda i: (0, 0)),
            pl.BlockSpec((bm, 1), lambda i: (i, 0)),
        ],
        out_specs=[
            pl.BlockSpec((bm, D), lambda i: (i, 0)),
            pl.BlockSpec((bm, 1), lambda i: (i, 0)),
        ],
        out_shape=[
            jax.ShapeDtypeStruct((N, D), jnp.float32),
            jax.ShapeDtypeStruct((N, 1), jnp.float32),
        ],
    )(x, W, b, deg)


def _layer2_tc(acc1, dis, W, b, bm):
    N, D = acc1.shape

    def body(a_ref, dis_ref, w_ref, b_ref, hs_ref):
        dis = dis_ref[...]
        h_in = jnp.maximum(a_ref[...] * dis, 0.0)
        h = jnp.dot(h_in, w_ref[...],
                    preferred_element_type=jnp.float32) + b_ref[...]
        hs_ref[...] = h * dis

    return pl.pallas_call(
        body,
        grid=(N // bm,),
        in_specs=[
            pl.BlockSpec((bm, D), lambda i: (i, 0)),
            pl.BlockSpec((bm, 1), lambda i: (i, 0)),
            pl.BlockSpec((D, D), lambda i: (0, 0)),
            pl.BlockSpec((1, D), lambda i: (0, 0)),
        ],
        out_specs=pl.BlockSpec((bm, D), lambda i: (i, 0)),
        out_shape=jax.ShapeDtypeStruct((N, D), jnp.float32),
    )(acc1, dis, W, b)


def _scale_tc(acc2, dis, bm):
    N, D = acc2.shape

    def body(a_ref, dis_ref, o_ref):
        o_ref[...] = a_ref[...] * dis_ref[...]

    return pl.pallas_call(
        body,
        grid=(N // bm,),
        in_specs=[
            pl.BlockSpec((bm, D), lambda i: (i, 0)),
            pl.BlockSpec((bm, 1), lambda i: (i, 0)),
        ],
        out_specs=pl.BlockSpec((bm, D), lambda i: (i, 0)),
        out_shape=jax.ShapeDtypeStruct((N, D), jnp.float32),
    )(acc2, dis)


def kernel(x, edge_index, W1, b1, W2, b2):
    N, D = x.shape
    E = edge_index.shape[1]
    assert E % (NSEG * L) == 0 and D % L == 0
    bm = 1000 if N % 1000 == 0 else 8

    src = edge_index[0].astype(jnp.int32)
    dst = edge_index[1].astype(jnp.int32)

    deg, srcc, dstc, counts = _make_prep(N, E)(src, dst)
    deg = deg.reshape(N, 1)

    b1r = b1.reshape(1, D)
    b2r = b2.reshape(1, D)

    edge_fn = _make_edge(N, D, E)

    hs1, dis = _layer1_tc(x, W1, b1r, deg, bm)
    acc1 = edge_fn(hs1, hs1.reshape(-1), srcc, dstc, counts).reshape(N, D)
    hs2 = _layer2_tc(acc1, dis, W2, b2r, bm)
    acc2 = edge_fn(hs2, hs2.reshape(-1), srcc, dstc, counts).reshape(N, D)
    return _scale_tc(acc2, dis, bm)


# speedup vs baseline: 2.7551x; 1.4150x over previous
import functools

import jax
import jax.numpy as jnp
from jax import lax
from jax.experimental import pallas as pl
from jax.experimental.pallas import tpu as pltpu
from jax.experimental.pallas import tpu_sc as plsc

NC = 2
NS = 16
NW = NC * NS
L = 16
CH = 80
NSEG = 16


def _mesh():
    return plsc.VectorSubcoreMesh(core_axis_name="c", subcore_axis_name="s",
                                  num_cores=NC, num_subcores=NS)


def _slab_rows(N):
    R = -(-N // NW)
    R = -(-R // L) * L
    return R


def _make_prep(N, E):
    R = _slab_rows(N)
    SEG = E // NSEG
    M = (SEG + CH - 1) // CH + 1
    CAP = M * CH

    @functools.partial(
        pl.kernel,
        out_type=(
            jax.ShapeDtypeStruct((N,), jnp.float32),
            jax.ShapeDtypeStruct((NW * NSEG * CAP,), jnp.int32),
            jax.ShapeDtypeStruct((NW * NSEG * CAP,), jnp.int32),
            jax.ShapeDtypeStruct((NW * NSEG * L,), jnp.int32),
        ),
        mesh=_mesh(),
        compiler_params=pltpu.CompilerParams(needs_layout_passes=False),
        scratch_types=[
            pltpu.VMEM((SEG,), jnp.int32),
            pltpu.VMEM((SEG,), jnp.int32),
            pltpu.VMEM((CAP,), jnp.int32),
            pltpu.VMEM((CAP,), jnp.int32),
            pltpu.VMEM((R + L,), jnp.float32),
            pltpu.VMEM((L,), jnp.int32),
        ],
    )
    def prep(src_hbm, dst_hbm, deg_hbm, srcc_hbm, dstc_hbm, cnt_hbm,
             sbuf, dbuf, scb, dcb, part, cntb):
        c = lax.axis_index("c")
        s = lax.axis_index("s")
        wid = c * NS + s
        base = wid * R

        zf = jnp.zeros((L,), jnp.float32)

        def zero_part(i, _):
            part[pl.ds(i * L, L)] = zf
            return 0
        lax.fori_loop(0, (R + L) // L, zero_part, 0)

        zi = jnp.zeros((L,), jnp.int32)
        dummy = jnp.full((L,), R, jnp.int32)
        ones = jnp.ones((L,), jnp.float32)

        def seg_body(k, _):
            pltpu.sync_copy(src_hbm.at[pl.ds(k * SEG, SEG)], sbuf)
            pltpu.sync_copy(dst_hbm.at[pl.ds(k * SEG, SEG)], dbuf)

            def prefill(i, _):
                scb[pl.ds(i * L, L)] = zi
                dcb[pl.ds(i * L, L)] = dummy
                return 0
            lax.fori_loop(0, CAP // L, prefill, 0)

            def scan_body(j, cnt):
                sv = sbuf[pl.ds(j * L, L)]
                dv = dbuf[pl.ds(j * L, L)]
                loc = dv - base
                mask = (loc >= 0) & (loc < R)
                locc = jnp.where(mask, loc, R)
                plsc.addupdate_scatter(part, [locc], ones, mask=mask)
                plsc.store_compressed(scb.at[pl.ds(cnt, L)], sv, mask=mask)
                plsc.store_compressed(dcb.at[pl.ds(cnt, L)], locc, mask=mask)
                return cnt + jnp.sum(mask.astype(jnp.int32))
            cnt = lax.fori_loop(0, SEG // L, scan_body, jnp.int32(0))

            cntb[...] = jnp.full((L,), cnt, jnp.int32)
            pltpu.sync_copy(cntb, cnt_hbm.at[pl.ds((wid * NSEG + k) * L, L)])
            pltpu.sync_copy(scb, srcc_hbm.at[pl.ds((wid * NSEG + k) * CAP, CAP)])
            pltpu.sync_copy(dcb, dstc_hbm.at[pl.ds((wid * NSEG + k) * CAP, CAP)])
            return 0
        lax.fori_loop(0, NSEG, seg_body, 0)

        def deg_body(i, _):
            part[pl.ds(i * L, L)] = part[pl.ds(i * L, L)] + 1.0
            return 0
        lax.fori_loop(0, R // L, deg_body, 0)

        rows = N - (NW - 1) * R

        @pl.when(wid < NW - 1)
        def _():
            pltpu.sync_copy(part.at[pl.ds(0, R)], deg_hbm.at[pl.ds(base, R)])

        @pl.when(wid == NW - 1)
        def _():
            pltpu.sync_copy(part.at[pl.ds(0, rows)], deg_hbm.at[pl.ds(base, rows)])

    return prep


def _make_edge(N, D, E):
    R = _slab_rows(N)
    SEG = E // NSEG
    M = (SEG + CH - 1) // CH + 1
    CAP = M * CH
    SLAB = R + 1

    @functools.partial(
        pl.kernel,
        out_type=jax.ShapeDtypeStruct((N * D,), jnp.float32),
        mesh=_mesh(),
        compiler_params=pltpu.CompilerParams(needs_layout_passes=False),
        scratch_types=[
            pltpu.VMEM((2 * CH,), jnp.int32),
            pltpu.VMEM((2 * CH,), jnp.int32),
            pltpu.VMEM((NSEG * L,), jnp.int32),
            pltpu.VMEM((CH, D), jnp.float32),
            pltpu.VMEM((CH, D), jnp.float32),
            pltpu.VMEM((SLAB * D,), jnp.float32),
            pltpu.SemaphoreType.DMA,
            pltpu.SemaphoreType.DMA,
        ],
    )
    def edge(hs_hbm, hsf_hbm, srcc_hbm, dstc_hbm, cnt_hbm, out_hbm,
             idxs, idxd, cb, gbuf0, gbuf1, slab, sem0, sem1):
        c = lax.axis_index("c")
        s = lax.axis_index("s")
        wid = c * NS + s
        base = wid * R
        rows = N - (NW - 1) * R

        pltpu.sync_copy(cnt_hbm.at[pl.ds(wid * NSEG * L, NSEG * L)], cb)

        @pl.when(wid < NW - 1)
        def _():
            pltpu.sync_copy(hsf_hbm.at[pl.ds(base * D, R * D)],
                            slab.at[pl.ds(0, R * D)])

        @pl.when(wid == NW - 1)
        def _():
            pltpu.sync_copy(hsf_hbm.at[pl.ds(base * D, rows * D)],
                            slab.at[pl.ds(0, rows * D)])

        gbufs = (gbuf0, gbuf1)
        sems = (sem0, sem1)

        def fetch(segbase, t, b):
            ebase = segbase + t * CH
            pltpu.sync_copy(srcc_hbm.at[pl.ds(ebase, CH)],
                            idxs.at[pl.ds(b * CH, CH)])
            pltpu.sync_copy(dstc_hbm.at[pl.ds(ebase, CH)],
                            idxd.at[pl.ds(b * CH, CH)])
            pltpu.async_copy(hs_hbm.at[idxs.at[pl.ds(b * CH, CH)]],
                             gbufs[b], sems[b])

        def accum(b):
            gbuf = gbufs[b]

            def group(g, _):
                dvec = idxd[pl.ds(b * CH + g * L, L)] * D
                for l in range(L):
                    rbase = dvec[l]
                    grow = g * L + l
                    for j in range(D // L):
                        plsc.addupdate(slab.at[pl.ds(rbase + j * L, L)],
                                       gbuf[grow, pl.ds(j * L, L)])
                return 0
            lax.fori_loop(0, CH // L, group, 0)

        def seg_body(k, _):
            cnt = cb[pl.ds(k * L, L)][0]
            nch = (cnt + CH - 1) // CH
            segbase = (wid * NSEG + k) * CAP

            @pl.when(nch > 0)
            def _():
                fetch(segbase, 0, 0)

            def chunk(t, _):
                b = lax.rem(t, 2)

                @pl.when((t + 1 < nch) & (b == 0))
                def _():
                    fetch(segbase, t + 1, 1)

                @pl.when((t + 1 < nch) & (b == 1))
                def _():
                    fetch(segbase, t + 1, 0)

                @pl.when(b == 0)
                def _():
                    pltpu.make_async_copy(
                        hs_hbm.at[idxs.at[pl.ds(0, CH)]], gbuf0, sem0).wait()
                    accum(0)

                @pl.when(b == 1)
                def _():
                    pltpu.make_async_copy(
                        hs_hbm.at[idxs.at[pl.ds(CH, CH)]], gbuf1, sem1).wait()
                    accum(1)
                return 0
            lax.fori_loop(0, nch, chunk, 0)
            return 0
        lax.fori_loop(0, NSEG, seg_body, 0)

        @pl.when(wid < NW - 1)
        def _():
            pltpu.sync_copy(slab.at[pl.ds(0, R * D)],
                            out_hbm.at[pl.ds(base * D, R * D)])

        @pl.when(wid == NW - 1)
        def _():
            pltpu.sync_copy(slab.at[pl.ds(0, rows * D)],
                            out_hbm.at[pl.ds(base * D, rows * D)])

    return edge


def _layer1_tc(x, W, b, deg, bm):
    N, D = x.shape

    def body(x_ref, w_ref, b_ref, deg_ref, hs_ref, dis_ref):
        dis = lax.rsqrt(deg_ref[...])
        h = jnp.dot(x_ref[...], w_ref[...],
                    preferred_element_type=jnp.float32) + b_ref[...]
        hs_ref[...] = h * dis
        dis_ref[...] = dis

    return pl.pallas_call(
        body,
        grid=(N // bm,),
        in_specs=[
            pl.BlockSpec((bm, D), lambda i: (i, 0)),
            pl.BlockSpec((D, D), lambda i: (0, 0)),
            pl.BlockSpec((1, D), lambda i: (0, 0)),
            pl.BlockSpec((bm, 1), lambda i: (i, 0)),
        ],
        out_specs=[
            pl.BlockSpec((bm, D), lambda i: (i, 0)),
            pl.BlockSpec((bm, 1), lambda i: (i, 0)),
        ],
        out_shape=[
            jax.ShapeDtypeStruct((N, D), jnp.float32),
            jax.ShapeDtypeStruct((N, 1), jnp.float32),
        ],
    )(x, W, b, deg)


def _layer2_tc(acc1, dis, W, b, bm):
    N, D = acc1.shape

    def body(a_ref, dis_ref, w_ref, b_ref, hs_ref):
        dis = dis_ref[...]
        h_in = jnp.maximum(a_ref[...] * dis, 0.0)
        h = jnp.dot(h_in, w_ref[...],
                    preferred_element_type=jnp.float32) + b_ref[...]
        hs_ref[...] = h * dis

    return pl.pallas_call(
        body,
        grid=(N // bm,),
        in_specs=[
            pl.BlockSpec((bm, D), lambda i: (i, 0)),
            pl.BlockSpec((bm, 1), lambda i: (i, 0)),
            pl.BlockSpec((D, D), lambda i: (0, 0)),
            pl.BlockSpec((1, D), lambda i: (0, 0)),
        ],
        out_specs=pl.BlockSpec((bm, D), lambda i: (i, 0)),
        out_shape=jax.ShapeDtypeStruct((N, D), jnp.float32),
    )(acc1, dis, W, b)


def _scale_tc(acc2, dis, bm):
    N, D = acc2.shape

    def body(a_ref, dis_ref, o_ref):
        o_ref[...] = a_ref[...] * dis_ref[...]

    return pl.pallas_call(
        body,
        grid=(N // bm,),
        in_specs=[
            pl.BlockSpec((bm, D), lambda i: (i, 0)),
            pl.BlockSpec((bm, 1), lambda i: (i, 0)),
        ],
        out_specs=pl.BlockSpec((bm, D), lambda i: (i, 0)),
        out_shape=jax.ShapeDtypeStruct((N, D), jnp.float32),
    )(acc2, dis)


def kernel(x, edge_index, W1, b1, W2, b2):
    N, D = x.shape
    E = edge_index.shape[1]
    assert E % (NSEG * L) == 0 and D % L == 0
    bm = 1000 if N % 1000 == 0 else 8

    src = edge_index[0].astype(jnp.int32)
    dst = edge_index[1].astype(jnp.int32)

    deg, srcc, dstc, counts = _make_prep(N, E)(src, dst)
    deg = deg.reshape(N, 1)

    b1r = b1.reshape(1, D)
    b2r = b2.reshape(1, D)

    edge_fn = _make_edge(N, D, E)

    hs1, dis = _layer1_tc(x, W1, b1r, deg, bm)
    acc1 = edge_fn(hs1, hs1.reshape(-1), srcc, dstc, counts).reshape(N, D)
    hs2 = _layer2_tc(acc1, dis, W2, b2r, bm)
    acc2 = edge_fn(hs2, hs2.reshape(-1), srcc, dstc, counts).reshape(N, D)
    return _scale_tc(acc2, dis, bm)
